# baseline (speedup 1.0000x reference)
import functools

import jax
import jax.numpy as jnp
from jax import lax
from jax.experimental import pallas as pl
from jax.experimental.pallas import tpu as pltpu
from jax.experimental.pallas import tpu_sc as plsc

N_GENES = 10000
WIDTH = 4
LATENT = 256
FAN_IN = 32
HIDDEN = N_GENES * WIDTH
BATCH = 256
NEG_SLOPE = 0.01

HIDDEN_PAD = 40960
GENES_PAD = HIDDEN_PAD // WIDTH
CHUNK = 256
N_CHUNKS = HIDDEN_PAD // CHUNK
NUM_WORKERS = 32
CHUNKS_PER_W = N_CHUNKS // NUM_WORKERS
G_B = CHUNK // WIDTH


def _sc_build(cm_hbm, wm_hbm, wt_hbm, cm_v, wm_v, buf):
    wid = lax.axis_index("s") * 2 + lax.axis_index("c")
    lane = lax.iota(jnp.int32, 16)
    for t in range(CHUNKS_PER_W):
        cid = wid * CHUNKS_PER_W + t
        pltpu.sync_copy(cm_hbm.at[pl.ds(cid * CHUNK * FAN_IN, CHUNK * FAN_IN)], cm_v)
        pltpu.sync_copy(wm_hbm.at[pl.ds(cid * CHUNK * FAN_IN, CHUNK * FAN_IN)], wm_v)

        def zbody(i, carry):
            buf[pl.ds(i * 16, 16)] = jnp.zeros((16,), jnp.float32)
            return carry

        lax.fori_loop(0, LATENT * CHUNK // 16, zbody, 0)

        def gbody(g, carry):
            node = g * 16 + lane
            for k in range(FAN_IN):
                c = cm_v[pl.ds(k * CHUNK + g * 16, 16)]
                w = wm_v[pl.ds(k * CHUNK + g * 16, 16)]
                plsc.addupdate_scatter(buf, [c * CHUNK + node], w)
            return carry

        lax.fori_loop(0, CHUNK // 16, gbody, 0)
        pltpu.sync_copy(buf, wt_hbm.at[cid])


H_B = 2 * CHUNK
GT_B = H_B // WIDTH


def _tc_body(f_ref, wt_ref, b1_ref, w2_ref, b2_ref, out_ref):
    h = jnp.concatenate(
        [jnp.dot(f_ref[...], wt_ref[0], preferred_element_type=jnp.float32),
         jnp.dot(f_ref[...], wt_ref[1], preferred_element_type=jnp.float32)],
        axis=1)
    h = h + b1_ref[...]
    h = jnp.where(h >= 0, h, NEG_SLOPE * h)
    hid_iota = lax.broadcasted_iota(jnp.int32, (H_B, GT_B), 0)
    gene_iota = lax.broadcasted_iota(jnp.int32, (H_B, GT_B), 1)
    pool = jnp.where(hid_iota // WIDTH == gene_iota,
                     w2_ref[...].reshape(H_B, 1), 0.0)
    out_ref[...] = jnp.dot(h, pool, preferred_element_type=jnp.float32) + b2_ref[...]


def kernel(features, w1, b1, w2, b2, conn1_row, conn1_col, conn2_row, conn2_col):
    del conn1_row, conn2_row, conn2_col
    pad_h = HIDDEN_PAD - HIDDEN
    cm = (jnp.pad(conn1_col, (0, pad_h * FAN_IN))
          .reshape(N_CHUNKS, CHUNK, FAN_IN).transpose(0, 2, 1).reshape(-1))
    wm = (jnp.pad(w1, (0, pad_h * FAN_IN))
          .reshape(N_CHUNKS, CHUNK, FAN_IN).transpose(0, 2, 1).reshape(-1))
    b1p = jnp.pad(b1, (0, pad_h)).reshape(1, HIDDEN_PAD)
    w2p = jnp.pad(w2, (0, pad_h)).reshape(1, HIDDEN_PAD)
    b2p = jnp.pad(b2, (0, GENES_PAD - N_GENES)).reshape(1, GENES_PAD)

    mesh = plsc.VectorSubcoreMesh(core_axis_name="c", subcore_axis_name="s")
    wt_flat = pl.kernel(
        _sc_build,
        out_type=jax.ShapeDtypeStruct((N_CHUNKS, LATENT * CHUNK), jnp.float32),
        mesh=mesh,
        scratch_types=[
            pltpu.VMEM((CHUNK * FAN_IN,), jnp.int32),
            pltpu.VMEM((CHUNK * FAN_IN,), jnp.float32),
            pltpu.VMEM((LATENT * CHUNK,), jnp.float32),
        ],
        compiler_params=pltpu.CompilerParams(needs_layout_passes=False),
    )(cm, wm)
    wt3 = wt_flat.reshape(N_CHUNKS, LATENT, CHUNK)

    out = pl.pallas_call(
        _tc_body,
        grid=(N_CHUNKS // 2,),
        in_specs=[
            pl.BlockSpec((BATCH, LATENT), lambda i: (0, 0)),
            pl.BlockSpec((2, LATENT, CHUNK), lambda i: (i, 0, 0)),
            pl.BlockSpec((1, H_B), lambda i: (0, i)),
            pl.BlockSpec((1, H_B), lambda i: (0, i)),
            pl.BlockSpec((1, GT_B), lambda i: (0, i)),
        ],
        out_specs=pl.BlockSpec((BATCH, GT_B), lambda i: (0, i)),
        out_shape=jax.ShapeDtypeStruct((BATCH, GENES_PAD), jnp.float32),
    )(features, wt3, b1p, w2p, b2p)
    return out[:, :N_GENES]


# speedup vs baseline: 1.3173x; 1.3173x over previous
import functools

import jax
import jax.numpy as jnp
from jax import lax
from jax.experimental import pallas as pl
from jax.experimental.pallas import tpu as pltpu
from jax.experimental.pallas import tpu_sc as plsc

N_GENES = 10000
WIDTH = 4
LATENT = 256
FAN_IN = 32
HIDDEN = N_GENES * WIDTH
BATCH = 256
NEG_SLOPE = 0.01

HIDDEN_PAD = 40960
GENES_PAD = HIDDEN_PAD // WIDTH
CHUNK = 256
N_CHUNKS = HIDDEN_PAD // CHUNK
NUM_WORKERS = 32
CHUNKS_PER_W = N_CHUNKS // NUM_WORKERS
G_B = CHUNK // WIDTH


def _sc_build(cm_hbm, wm_hbm, wt_hbm, cm_v, wm_v, buf):
    wid = lax.axis_index("s") * 2 + lax.axis_index("c")
    lane = lax.iota(jnp.int32, 16)
    for t in range(CHUNKS_PER_W):
        cid = wid * CHUNKS_PER_W + t
        pltpu.sync_copy(cm_hbm.at[pl.ds(cid * CHUNK * FAN_IN, CHUNK * FAN_IN)], cm_v)
        pltpu.sync_copy(wm_hbm.at[pl.ds(cid * CHUNK * FAN_IN, CHUNK * FAN_IN)], wm_v)

        zero16 = jnp.zeros((16,), jnp.float32)

        def zbody(i, carry):
            for j in range(16):
                buf[pl.ds(i * 256 + j * 16, 16)] = zero16
            return carry

        lax.fori_loop(0, LATENT * CHUNK // 256, zbody, 0)

        def gbody(g, carry):
            node = g * 16 + lane
            nnz_base = node * FAN_IN
            for k in range(FAN_IN):
                c = plsc.load_gather(cm_v, [nnz_base + k])
                w = plsc.load_gather(wm_v, [nnz_base + k])
                plsc.addupdate_scatter(buf, [c * CHUNK + node], w)
            return carry

        lax.fori_loop(0, CHUNK // 16, gbody, 0)
        pltpu.sync_copy(buf, wt_hbm.at[cid])


H_B = 2 * CHUNK
GT_B = H_B // WIDTH


def _tc_body(f_ref, wt_ref, b1_ref, w2_ref, b2_ref, out_ref):
    h = jnp.concatenate(
        [jnp.dot(f_ref[...], wt_ref[0], preferred_element_type=jnp.float32),
         jnp.dot(f_ref[...], wt_ref[1], preferred_element_type=jnp.float32)],
        axis=1)
    h = h + b1_ref[...]
    h = jnp.where(h >= 0, h, NEG_SLOPE * h)
    hid_iota = lax.broadcasted_iota(jnp.int32, (H_B, GT_B), 0)
    gene_iota = lax.broadcasted_iota(jnp.int32, (H_B, GT_B), 1)
    pool = jnp.where(hid_iota // WIDTH == gene_iota,
                     w2_ref[...].reshape(H_B, 1), 0.0)
    out_ref[...] = jnp.dot(h, pool, preferred_element_type=jnp.float32) + b2_ref[...]


def kernel(features, w1, b1, w2, b2, conn1_row, conn1_col, conn2_row, conn2_col):
    del conn1_row, conn2_row, conn2_col
    pad_h = HIDDEN_PAD - HIDDEN
    cm = jnp.pad(conn1_col, (0, pad_h * FAN_IN))
    wm = jnp.pad(w1, (0, pad_h * FAN_IN))
    b1p = jnp.pad(b1, (0, pad_h)).reshape(1, HIDDEN_PAD)
    w2p = jnp.pad(w2, (0, pad_h)).reshape(1, HIDDEN_PAD)
    b2p = jnp.pad(b2, (0, GENES_PAD - N_GENES)).reshape(1, GENES_PAD)

    mesh = plsc.VectorSubcoreMesh(core_axis_name="c", subcore_axis_name="s")
    wt_flat = pl.kernel(
        _sc_build,
        out_type=jax.ShapeDtypeStruct((N_CHUNKS, LATENT * CHUNK), jnp.float32),
        mesh=mesh,
        scratch_types=[
            pltpu.VMEM((CHUNK * FAN_IN,), jnp.int32),
            pltpu.VMEM((CHUNK * FAN_IN,), jnp.float32),
            pltpu.VMEM((LATENT * CHUNK,), jnp.float32),
        ],
        compiler_params=pltpu.CompilerParams(needs_layout_passes=False),
    )(cm, wm)
    wt3 = wt_flat.reshape(N_CHUNKS, LATENT, CHUNK)

    out = pl.pallas_call(
        _tc_body,
        grid=(N_CHUNKS // 2,),
        in_specs=[
            pl.BlockSpec((BATCH, LATENT), lambda i: (0, 0)),
            pl.BlockSpec((2, LATENT, CHUNK), lambda i: (i, 0, 0)),
            pl.BlockSpec((1, H_B), lambda i: (0, i)),
            pl.BlockSpec((1, H_B), lambda i: (0, i)),
            pl.BlockSpec((1, GT_B), lambda i: (0, i)),
        ],
        out_specs=pl.BlockSpec((BATCH, GT_B), lambda i: (0, i)),
        out_shape=jax.ShapeDtypeStruct((BATCH, GENES_PAD), jnp.float32),
    )(features, wt3, b1p, w2p, b2p)
    return out[:, :N_GENES]
